# Initial kernel scaffold; baseline (speedup 1.0000x reference)
import jax
import jax.numpy as jnp
from jax.experimental import pallas as pl


def kernel(u_idx, r_idx, v_idx, E, Wu, rv, bs, bo, E1, Wu1, rv1):
    raise NotImplementedError("write your pallas kernel here")


# speedup vs baseline: 2.7879x; 2.7879x over previous
import functools

import jax
import jax.numpy as jnp
from jax import lax
from jax.experimental import pallas as pl
from jax.experimental.pallas import tpu as pltpu
from jax.experimental.pallas import tpu_sc as plsc

NE = 100000
NR = 1000
D = 128
B = 16384

NC = 2
NS = 16
L = 16
NW = NC * NS
BPW = B // NW
CH = 64
NCHUNK = BPW // CH
NGRP = CH // L


def _rsqrt(p):
    i = plsc.bitcast(p, jnp.int32)
    i = jnp.int32(0x5F3759DF) - lax.shift_right_logical(i, 1)
    y = plsc.bitcast(i, jnp.float32)
    for _ in range(4):
        y = y * (1.5 - 0.5 * p * y * y)
    return y


def _sc_body(u_idx, v_idx, r_idx, E, Wu, rv, bs, bo, E1, rv1, out,
             idx_u, idx_v, idx_r, bs_b, bo_b,
             u_r, v_r, u1_r, v1_r, ru_r, rvv_r, rv1_r, out_v, sem):
    wid = lax.axis_index("s") * NC + lax.axis_index("c")
    base = wid * BPW
    lane = lax.iota(jnp.int32, 16)
    zero = jnp.zeros((16,), jnp.float32)

    for c in range(NCHUNK):
        off = base + c * CH
        pltpu.sync_copy(u_idx.at[pl.ds(off, CH)], idx_u.at[c])
        pltpu.sync_copy(v_idx.at[pl.ds(off, CH)], idx_v.at[c])
        pltpu.sync_copy(r_idx.at[pl.ds(off, CH)], idx_r.at[c])
        cps = [
            pltpu.async_copy(E.at[idx_u.at[c]], u_r, sem),
            pltpu.async_copy(E.at[idx_v.at[c]], v_r, sem),
            pltpu.async_copy(E1.at[idx_u.at[c]], u1_r, sem),
            pltpu.async_copy(E1.at[idx_v.at[c]], v1_r, sem),
            pltpu.async_copy(Wu.at[idx_r.at[c]], ru_r, sem),
            pltpu.async_copy(rv.at[idx_r.at[c]], rvv_r, sem),
            pltpu.async_copy(rv1.at[idx_r.at[c]], rv1_r, sem),
            pltpu.async_copy(bs.at[idx_u.at[c]], bs_b.at[c], sem),
            pltpu.async_copy(bo.at[idx_v.at[c]], bo_b.at[c], sem),
        ]
        for cp in cps:
            cp.wait()

        for g in range(NGRP):
            rows = lane + jnp.int32(g * L)

            def dbody(d, carry, rows=rows):
                su, sa, dot, e = carry
                col = jnp.bitwise_and(d + lane, D - 1)
                u = plsc.load_gather(u_r, [rows, col])
                ru = plsc.load_gather(ru_r, [rows, col])
                uw = u * ru
                v = plsc.load_gather(v_r, [rows, col])
                rvv = plsc.load_gather(rvv_r, [rows, col])
                a = v + rvv
                su = su + uw * uw
                sa = sa + a * a
                dot = dot + uw * a
                u1 = plsc.load_gather(u1_r, [rows, col])
                v1 = plsc.load_gather(v1_r, [rows, col])
                rv1v = plsc.load_gather(rv1_r, [rows, col])
                d1 = u1 * ru - v1 - rv1v
                e = e + d1 * d1
                return (su, sa, dot, e)

            su, sa, dot, e = lax.fori_loop(
                0, D, dbody, (zero, zero, zero, zero))
            p = (su + 1.0) * (sa + 1.0)
            s = p * _rsqrt(p)
            res = (2.0 + 2.0 * dot - 2.0 * s - e
                   + bs_b[c, pl.ds(g * L, L)] + bo_b[c, pl.ds(g * L, L)])
            out_v[pl.ds(c * CH + g * L, L)] = res

    pltpu.sync_copy(out_v, out.at[pl.ds(base, BPW)])


@functools.partial(jax.jit, static_argnames=())
def _mu_rel_sc(u_idx, r_idx, v_idx, E, Wu, rv, bs, bo, E1, rv1):
    mesh = plsc.VectorSubcoreMesh(core_axis_name="c", subcore_axis_name="s")
    kern = pl.kernel(
        _sc_body,
        out_type=jax.ShapeDtypeStruct((B,), jnp.float32),
        mesh=mesh,
        scratch_types=[
            pltpu.VMEM((NCHUNK, CH), jnp.int32),
            pltpu.VMEM((NCHUNK, CH), jnp.int32),
            pltpu.VMEM((NCHUNK, CH), jnp.int32),
            pltpu.VMEM((NCHUNK, CH), jnp.float32),
            pltpu.VMEM((NCHUNK, CH), jnp.float32),
            pltpu.VMEM((CH, D), jnp.float32),
            pltpu.VMEM((CH, D), jnp.float32),
            pltpu.VMEM((CH, D), jnp.float32),
            pltpu.VMEM((CH, D), jnp.float32),
            pltpu.VMEM((CH, D), jnp.float32),
            pltpu.VMEM((CH, D), jnp.float32),
            pltpu.VMEM((CH, D), jnp.float32),
            pltpu.VMEM((BPW,), jnp.float32),
            pltpu.SemaphoreType.DMA,
        ],
        compiler_params=pltpu.CompilerParams(
            use_tc_tiling_on_sc=False, needs_layout_passes=False),
    )
    return kern(u_idx, v_idx, r_idx, E, Wu, rv, bs, bo, E1, rv1)


def kernel(u_idx, r_idx, v_idx, E, Wu, rv, bs, bo, E1, Wu1, rv1):
    del Wu1
    return _mu_rel_sc(u_idx, r_idx, v_idx, E, Wu, rv, bs, bo, E1, rv1)


# speedup vs baseline: 4.0798x; 1.4634x over previous
import functools

import jax
import jax.numpy as jnp
from jax import lax
from jax.experimental import pallas as pl
from jax.experimental.pallas import tpu as pltpu
from jax.experimental.pallas import tpu_sc as plsc

NE = 100000
NR = 1000
D = 128
B = 16384

NC = 2
NS = 16
L = 16
NW = NC * NS
BPW = B // NW
CH = 64
NCHUNK = BPW // CH
NGRP = CH // L


def _rsqrt(p):
    i = plsc.bitcast(p, jnp.int32)
    i = jnp.int32(0x5F3759DF) - lax.shift_right_logical(i, 1)
    y = plsc.bitcast(i, jnp.float32)
    for _ in range(4):
        y = y * (1.5 - 0.5 * p * y * y)
    return y


def _sc_body(u_idx, v_idx, r_idx, E, Wu, rv, bs, bo, E1, rv1, out,
             idx_u, idx_v, idx_r, bs_b, bo_b,
             u_r, v_r, u1_r, v1_r, ru_r, rvv_r, rv1_r, out_v, sem):
    wid = lax.axis_index("s") * NC + lax.axis_index("c")
    base = wid * BPW
    lane = lax.iota(jnp.int32, 16)
    zero = jnp.zeros((16,), jnp.float32)

    pltpu.sync_copy(u_idx.at[pl.ds(base, BPW)], idx_u)
    pltpu.sync_copy(v_idx.at[pl.ds(base, BPW)], idx_v)
    pltpu.sync_copy(r_idx.at[pl.ds(base, BPW)], idx_r)

    def issue(c, p):
        iu = idx_u.at[pl.ds(c * CH, CH)]
        iv = idx_v.at[pl.ds(c * CH, CH)]
        ir = idx_r.at[pl.ds(c * CH, CH)]
        s = sem.at[p]
        return [
            pltpu.async_copy(E.at[iu], u_r.at[p], s),
            pltpu.async_copy(E.at[iv], v_r.at[p], s),
            pltpu.async_copy(E1.at[iu], u1_r.at[p], s),
            pltpu.async_copy(E1.at[iv], v1_r.at[p], s),
            pltpu.async_copy(Wu.at[ir], ru_r.at[p], s),
            pltpu.async_copy(rv.at[ir], rvv_r.at[p], s),
            pltpu.async_copy(rv1.at[ir], rv1_r.at[p], s),
            pltpu.async_copy(bs.at[iu], bs_b.at[p], s),
            pltpu.async_copy(bo.at[iv], bo_b.at[p], s),
        ]

    pending = {0: issue(0, 0)}
    for c in range(NCHUNK):
        p = c & 1
        if c + 1 < NCHUNK:
            pending[c + 1] = issue(c + 1, (c + 1) & 1)
        for cp in pending.pop(c):
            cp.wait()

        for g in range(NGRP):
            rows = lane + jnp.int32(g * L)

            def dbody(d, carry, rows=rows, p=p):
                su, sa, dot, e = carry
                col = jnp.bitwise_and(d + lane, D - 1)
                u = plsc.load_gather(u_r.at[p], [rows, col])
                ru = plsc.load_gather(ru_r.at[p], [rows, col])
                uw = u * ru
                v = plsc.load_gather(v_r.at[p], [rows, col])
                rvv = plsc.load_gather(rvv_r.at[p], [rows, col])
                a = v + rvv
                su = su + uw * uw
                sa = sa + a * a
                dot = dot + uw * a
                u1 = plsc.load_gather(u1_r.at[p], [rows, col])
                v1 = plsc.load_gather(v1_r.at[p], [rows, col])
                rv1v = plsc.load_gather(rv1_r.at[p], [rows, col])
                d1 = u1 * ru - v1 - rv1v
                e = e + d1 * d1
                return (su, sa, dot, e)

            su, sa, dot, e = lax.fori_loop(
                0, D, dbody, (zero, zero, zero, zero))
            q = (su + 1.0) * (sa + 1.0)
            sq = q * _rsqrt(q)
            res = (2.0 + 2.0 * dot - 2.0 * sq - e
                   + bs_b[p, pl.ds(g * L, L)] + bo_b[p, pl.ds(g * L, L)])
            out_v[pl.ds(c * CH + g * L, L)] = res

    pltpu.sync_copy(out_v, out.at[pl.ds(base, BPW)])


@jax.jit
def _mu_rel_sc(u_idx, r_idx, v_idx, E, Wu, rv, bs, bo, E1, rv1):
    mesh = plsc.VectorSubcoreMesh(core_axis_name="c", subcore_axis_name="s")
    kern = pl.kernel(
        _sc_body,
        out_type=jax.ShapeDtypeStruct((B,), jnp.float32),
        mesh=mesh,
        scratch_types=[
            pltpu.VMEM((BPW,), jnp.int32),
            pltpu.VMEM((BPW,), jnp.int32),
            pltpu.VMEM((BPW,), jnp.int32),
            pltpu.VMEM((2, CH), jnp.float32),
            pltpu.VMEM((2, CH), jnp.float32),
            pltpu.VMEM((2, CH, D), jnp.float32),
            pltpu.VMEM((2, CH, D), jnp.float32),
            pltpu.VMEM((2, CH, D), jnp.float32),
            pltpu.VMEM((2, CH, D), jnp.float32),
            pltpu.VMEM((2, CH, D), jnp.float32),
            pltpu.VMEM((2, CH, D), jnp.float32),
            pltpu.VMEM((2, CH, D), jnp.float32),
            pltpu.VMEM((BPW,), jnp.float32),
            pltpu.SemaphoreType.DMA((2,)),
        ],
        compiler_params=pltpu.CompilerParams(
            use_tc_tiling_on_sc=False, needs_layout_passes=False),
    )
    return kern(u_idx, v_idx, r_idx, E, Wu, rv, bs, bo, E1, rv1)


def kernel(u_idx, r_idx, v_idx, E, Wu, rv, bs, bo, E1, Wu1, rv1):
    del Wu1
    return _mu_rel_sc(u_idx, r_idx, v_idx, E, Wu, rv, bs, bo, E1, rv1)


# speedup vs baseline: 4.5546x; 1.1164x over previous
import functools

import jax
import jax.numpy as jnp
from jax import lax
from jax.experimental import pallas as pl
from jax.experimental.pallas import tpu as pltpu
from jax.experimental.pallas import tpu_sc as plsc

NE = 100000
NR = 1000
D = 128
B = 16384

NC = 2
NS = 16
L = 16
NW = NC * NS
BPW = B // NW
CH = 64
NCHUNK = BPW // CH
NGRP = CH // L


def _rsqrt(p):
    i = plsc.bitcast(p, jnp.int32)
    i = jnp.int32(0x5F3759DF) - lax.shift_right_logical(i, 1)
    y = plsc.bitcast(i, jnp.float32)
    for _ in range(4):
        y = y * (1.5 - 0.5 * p * y * y)
    return y


def _sc_body(u_idx, v_idx, r_idx, E, Wu, rv, bs, bo, E1, rv1, out,
             idx_u, idx_v, idx_r, bs_b, bo_b,
             u_r, v_r, u1_r, v1_r, ru_r, rvv_r, rv1_r, out_v, sem):
    wid = lax.axis_index("s") * NC + lax.axis_index("c")
    base = wid * BPW
    lane = lax.iota(jnp.int32, 16)
    zero = jnp.zeros((16,), jnp.float32)

    pltpu.sync_copy(u_idx.at[pl.ds(base, BPW)], idx_u)
    pltpu.sync_copy(v_idx.at[pl.ds(base, BPW)], idx_v)
    pltpu.sync_copy(r_idx.at[pl.ds(base, BPW)], idx_r)

    def bufs(p):
        return (u_r.at[p], v_r.at[p], u1_r.at[p], v1_r.at[p],
                ru_r.at[p], rvv_r.at[p], rv1_r.at[p],
                bs_b.at[p], bo_b.at[p])

    def srcs(c):
        iu = idx_u.at[pl.ds(c * CH, CH)]
        iv = idx_v.at[pl.ds(c * CH, CH)]
        ir = idx_r.at[pl.ds(c * CH, CH)]
        return (E.at[iu], E.at[iv], E1.at[iu], E1.at[iv],
                Wu.at[ir], rv.at[ir], rv1.at[ir], bs.at[iu], bo.at[iv])

    def issue(c, p):
        for s, b in zip(srcs(c), bufs(p)):
            pltpu.async_copy(s, b, sem.at[p])

    def drain(c, p):
        for s, b in zip(srcs(c), bufs(p)):
            pltpu.make_async_copy(s, b, sem.at[p]).wait()

    def compute(c, p):
        u_b, v_b, u1_b, v1_b, ru_b, rvv_b, rv1_b, bs_bp, bo_bp = bufs(p)

        def gbody(g, _):
            rows = lane + g * L

            def dbody(d, carry):
                su, sa, dot, e = carry
                col = jnp.bitwise_and(d + lane, D - 1)
                u = plsc.load_gather(u_b, [rows, col])
                ru = plsc.load_gather(ru_b, [rows, col])
                uw = u * ru
                v = plsc.load_gather(v_b, [rows, col])
                rvv = plsc.load_gather(rvv_b, [rows, col])
                a = v + rvv
                su = su + uw * uw
                sa = sa + a * a
                dot = dot + uw * a
                u1 = plsc.load_gather(u1_b, [rows, col])
                v1 = plsc.load_gather(v1_b, [rows, col])
                rv1v = plsc.load_gather(rv1_b, [rows, col])
                d1 = u1 * ru - v1 - rv1v
                e = e + d1 * d1
                return (su, sa, dot, e)

            su, sa, dot, e = lax.fori_loop(
                0, D, dbody, (zero, zero, zero, zero))
            q = (su + 1.0) * (sa + 1.0)
            sq = q * _rsqrt(q)
            res = (2.0 + 2.0 * dot - 2.0 * sq - e
                   + bs_bp[pl.ds(g * L, L)] + bo_bp[pl.ds(g * L, L)])
            out_v[pl.ds(c * CH + g * L, L)] = res
            return 0

        lax.fori_loop(0, NGRP, gbody, 0)

    issue(0, 0)

    def pair_body(t, _):
        c0 = 2 * t
        c1 = c0 + 1
        issue(c1, 1)
        drain(c0, 0)
        compute(c0, 0)

        @pl.when(t < NCHUNK // 2 - 1)
        def _():
            issue(c0 + 2, 0)

        drain(c1, 1)
        compute(c1, 1)
        return 0

    lax.fori_loop(0, NCHUNK // 2, pair_body, 0)

    pltpu.sync_copy(out_v, out.at[pl.ds(base, BPW)])


@jax.jit
def _mu_rel_sc(u_idx, r_idx, v_idx, E, Wu, rv, bs, bo, E1, rv1):
    mesh = plsc.VectorSubcoreMesh(core_axis_name="c", subcore_axis_name="s")
    kern = pl.kernel(
        _sc_body,
        out_type=jax.ShapeDtypeStruct((B,), jnp.float32),
        mesh=mesh,
        scratch_types=[
            pltpu.VMEM((BPW,), jnp.int32),
            pltpu.VMEM((BPW,), jnp.int32),
            pltpu.VMEM((BPW,), jnp.int32),
            pltpu.VMEM((2, CH), jnp.float32),
            pltpu.VMEM((2, CH), jnp.float32),
            pltpu.VMEM((2, CH, D), jnp.float32),
            pltpu.VMEM((2, CH, D), jnp.float32),
            pltpu.VMEM((2, CH, D), jnp.float32),
            pltpu.VMEM((2, CH, D), jnp.float32),
            pltpu.VMEM((2, CH, D), jnp.float32),
            pltpu.VMEM((2, CH, D), jnp.float32),
            pltpu.VMEM((2, CH, D), jnp.float32),
            pltpu.VMEM((BPW,), jnp.float32),
            pltpu.SemaphoreType.DMA((2,)),
        ],
        compiler_params=pltpu.CompilerParams(
            use_tc_tiling_on_sc=False, needs_layout_passes=False),
    )
    return kern(u_idx, v_idx, r_idx, E, Wu, rv, bs, bo, E1, rv1)


def kernel(u_idx, r_idx, v_idx, E, Wu, rv, bs, bo, E1, Wu1, rv1):
    del Wu1
    return _mu_rel_sc(u_idx, r_idx, v_idx, E, Wu, rv, bs, bo, E1, rv1)


# speedup vs baseline: 4.5577x; 1.0007x over previous
import functools

import jax
import jax.numpy as jnp
from jax import lax
from jax.experimental import pallas as pl
from jax.experimental.pallas import tpu as pltpu
from jax.experimental.pallas import tpu_sc as plsc

NE = 100000
NR = 1000
D = 128
B = 16384

NC = 2
NS = 16
L = 16
NW = NC * NS
BPW = B // NW
CH = 64
NCHUNK = BPW // CH
NGRP = CH // L


def _rsqrt(p):
    i = plsc.bitcast(p, jnp.int32)
    i = jnp.int32(0x5F3759DF) - lax.shift_right_logical(i, 1)
    y = plsc.bitcast(i, jnp.float32)
    for _ in range(4):
        y = y * (1.5 - 0.5 * p * y * y)
    return y


def _sc_body(u_idx, v_idx, r_idx, E, Wu, rv, bs, bo, E1, rv1, out,
             idx_u, idx_v, idx_r, bs_b, bo_b,
             u_r, v_r, u1_r, v1_r, ru_r, rvv_r, rv1_r, out_v, sem):
    wid = lax.axis_index("s") * NC + lax.axis_index("c")
    base = wid * BPW
    lane = lax.iota(jnp.int32, 16)
    zero = jnp.zeros((16,), jnp.float32)

    pltpu.sync_copy(u_idx.at[pl.ds(base, BPW)], idx_u)
    pltpu.sync_copy(v_idx.at[pl.ds(base, BPW)], idx_v)
    pltpu.sync_copy(r_idx.at[pl.ds(base, BPW)], idx_r)

    def bufs(p):
        return (u_r.at[p], v_r.at[p], u1_r.at[p], v1_r.at[p],
                ru_r.at[p], rvv_r.at[p], rv1_r.at[p],
                bs_b.at[p], bo_b.at[p])

    def srcs(c):
        iu = idx_u.at[pl.ds(c * CH, CH)]
        iv = idx_v.at[pl.ds(c * CH, CH)]
        ir = idx_r.at[pl.ds(c * CH, CH)]
        return (E.at[iu], E.at[iv], E1.at[iu], E1.at[iv],
                Wu.at[ir], rv.at[ir], rv1.at[ir], bs.at[iu], bo.at[iv])

    def issue(c, p):
        for s, b in zip(srcs(c), bufs(p)):
            pltpu.async_copy(s, b, sem.at[p])

    def drain(c, p):
        for s, b in zip(srcs(c), bufs(p)):
            pltpu.make_async_copy(s, b, sem.at[p]).wait()

    def compute(c, p):
        u_b, v_b, u1_b, v1_b, ru_b, rvv_b, rv1_b, bs_bp, bo_bp = bufs(p)

        def gbody(g, _):
            rows = lane + g * L

            def dbody(d, carry):
                su, sa, dot, e = carry
                col = jnp.bitwise_and(d + lane, D - 1)
                u = plsc.load_gather(u_b, [rows, col])
                ru = plsc.load_gather(ru_b, [rows, col])
                uw = u * ru
                v = plsc.load_gather(v_b, [rows, col])
                rvv = plsc.load_gather(rvv_b, [rows, col])
                a = v + rvv
                su = su + uw * uw
                sa = sa + a * a
                dot = dot + uw * a
                u1 = plsc.load_gather(u1_b, [rows, col])
                v1 = plsc.load_gather(v1_b, [rows, col])
                rv1v = plsc.load_gather(rv1_b, [rows, col])
                d1 = u1 * ru - v1 - rv1v
                e = e + d1 * d1
                return (su, sa, dot, e)

            su, sa, dot, e = lax.fori_loop(
                0, D, dbody, (zero, zero, zero, zero))
            q = (su + 1.0) * (sa + 1.0)
            sq = q * _rsqrt(q)
            res = (2.0 + 2.0 * dot - 2.0 * sq - e
                   + bs_bp[pl.ds(g * L, L)] + bo_bp[pl.ds(g * L, L)])
            out_v[pl.ds(c * CH + g * L, L)] = res
            return 0

        lax.fori_loop(0, NGRP, gbody, 0)

    issue(0, 0)

    def pair_body(t, _):
        c0 = 2 * t
        c1 = c0 + 1
        issue(c1, 1)
        drain(c0, 0)
        compute(c0, 0)

        @pl.when(t < NCHUNK // 2 - 1)
        def _():
            issue(c0 + 2, 0)

        drain(c1, 1)
        compute(c1, 1)
        return 0

    lax.fori_loop(0, NCHUNK // 2, pair_body, 0)

    pltpu.sync_copy(out_v, out.at[pl.ds(base, BPW)])


@jax.jit
def _mu_rel_sc(u_idx, r_idx, v_idx, E, Wu, rv, bs, bo, E1, rv1):
    mesh = plsc.VectorSubcoreMesh(core_axis_name="c", subcore_axis_name="s")
    kern = pl.kernel(
        _sc_body,
        out_type=jax.ShapeDtypeStruct((B,), jnp.float32),
        mesh=mesh,
        scratch_types=[
            pltpu.VMEM((BPW,), jnp.int32),
            pltpu.VMEM((BPW,), jnp.int32),
            pltpu.VMEM((BPW,), jnp.int32),
            pltpu.VMEM((2, CH), jnp.float32),
            pltpu.VMEM((2, CH), jnp.float32),
            pltpu.VMEM((2, CH, D), jnp.float32),
            pltpu.VMEM((2, CH, D), jnp.float32),
            pltpu.VMEM((2, CH, D), jnp.float32),
            pltpu.VMEM((2, CH, D), jnp.float32),
            pltpu.VMEM((2, CH, D), jnp.float32),
            pltpu.VMEM((2, CH, D), jnp.float32),
            pltpu.VMEM((2, CH, D), jnp.float32),
            pltpu.VMEM((BPW,), jnp.float32),
            pltpu.SemaphoreType.DMA((2,)),
        ],
        compiler_params=pltpu.CompilerParams(
            use_tc_tiling_on_sc=False, needs_layout_passes=False,
            skip_device_barrier=True, disable_bounds_checks=True),
    )
    return kern(u_idx, v_idx, r_idx, E, Wu, rv, bs, bo, E1, rv1)


def kernel(u_idx, r_idx, v_idx, E, Wu, rv, bs, bo, E1, Wu1, rv1):
    del Wu1
    return _mu_rel_sc(u_idx, r_idx, v_idx, E, Wu, rv, bs, bo, E1, rv1)
